# baseline (speedup 1.0000x reference)
import jax
import jax.numpy as jnp
from jax import lax
from jax.experimental import pallas as pl
from jax.experimental.pallas import tpu as pltpu
from jax.experimental.pallas import tpu_sc as plsc

M = 1_000_000
D = 64
B = 16384
TBL = 1 << 20
NC, NS = 2, 16
NW = NC * NS
SB = B // NW
IB = B // NS
ROUNDS = 3
FILL = 8192
REG = TBL // NS


def _body(mem2_hbm, idx_hbm, val2_hbm, samp_hbm, out2_hbm,
          tbl_sh, fill_v, idxs2, jv2, cur2, tgt2,
          samp2, p2, spair2, ppair2, rows_a, rows_b, outbuf):
    c = lax.axis_index("c")
    s = lax.axis_index("s")
    wid = s * NC + c
    ii16 = lax.iota(jnp.int32, 16)
    neg1 = jnp.full((16,), -1, jnp.int32)

    def _fill(i, _):
        fill_v[pl.ds(i * 16, 16)] = neg1
        return _
    lax.fori_loop(0, FILL // 16, _fill, 0)
    for b in range(REG // FILL):
        pltpu.sync_copy(fill_v, tbl_sh.at[pl.ds(s * REG + b * FILL, FILL)])

    for q in range(IB // 128):
        pltpu.sync_copy(idx_hbm.at[pl.ds(s * IB + q * 128, 128)], idxs2.at[q])
        for t in range(8):
            jv2[q, pl.ds(t * 16, 16)] = s * IB + q * 128 + t * 16 + ii16
    plsc.subcore_barrier()

    dump_cell = jnp.full((16,), M, jnp.int32) + wid
    for r in range(ROUNDS):
        for q in range(IB // 128):
            if r == 0:
                src = idxs2.at[q]
            else:
                pltpu.sync_copy(tbl_sh.at[idxs2.at[q]], cur2.at[q])
                for t in range(8):
                    cu = cur2[q, pl.ds(t * 16, 16)]
                    jj = jv2[q, pl.ds(t * 16, 16)]
                    ix = idxs2[q, pl.ds(t * 16, 16)]
                    tgt2[q, pl.ds(t * 16, 16)] = jnp.where(
                        cu < jj, ix, dump_cell)
                src = tgt2.at[q]
            pltpu.sync_copy(jv2.at[q], tbl_sh.at[src])
        plsc.subcore_barrier()

    for q in range(SB // 128):
        pltpu.sync_copy(samp_hbm.at[pl.ds(wid * SB + q * 128, 128)],
                        samp2.at[q])
        pltpu.sync_copy(tbl_sh.at[samp2.at[q]], p2.at[q])
        for t in range(8):
            sv = samp2[q, pl.ds(t * 16, 16)]
            pv = p2[q, pl.ds(t * 16, 16)]
            spair2[q, pl.ds(t * 16, 16)] = lax.shift_right_logical(sv, 1)
            ppair2[q, pl.ds(t * 16, 16)] = lax.shift_right_logical(
                jnp.maximum(pv, 0), 1)
        pltpu.sync_copy(mem2_hbm.at[spair2.at[q]], rows_a)
        pltpu.sync_copy(val2_hbm.at[ppair2.at[q]], rows_b)

        def _grp(g, _):
            sv16 = samp2[q, pl.ds(g * 16, 16)]
            pv16 = p2[q, pl.ds(g * 16, 16)]
            for l in range(16):
                i = g * 16 + l
                sv = sv16[l]
                pv = pv16[l]
                hit = pv >= 0
                sh = (sv & 1) * 64
                ph = (jnp.maximum(pv, 0) & 1) * 64
                orow = g * 8 + l // 2
                ocol = (l % 2) * 64
                for t in range(4):
                    a_t = rows_a[i, pl.ds(sh + t * 16, 16)]
                    b_t = rows_b[i, pl.ds(ph + t * 16, 16)]
                    outbuf[orow, pl.ds(ocol + t * 16, 16)] = jnp.where(
                        hit, b_t, a_t)
            return _
        lax.fori_loop(0, 8, _grp, 0)
        pltpu.sync_copy(outbuf, out2_hbm.at[pl.ds(wid * (SB // 2) + q * 64,
                                                  64)])


def _build():
    mesh = plsc.VectorSubcoreMesh(core_axis_name="c", subcore_axis_name="s")
    return pl.kernel(
        _body,
        out_type=jax.ShapeDtypeStruct((B // 2, 2 * D), jnp.float32),
        mesh=mesh,
        scratch_types=[
            pltpu.VMEM_SHARED((TBL,), jnp.int32),
            pltpu.VMEM((FILL,), jnp.int32),
            pltpu.VMEM((IB // 128, 128), jnp.int32),
            pltpu.VMEM((IB // 128, 128), jnp.int32),
            pltpu.VMEM((IB // 128, 128), jnp.int32),
            pltpu.VMEM((IB // 128, 128), jnp.int32),
            pltpu.VMEM((SB // 128, 128), jnp.int32),
            pltpu.VMEM((SB // 128, 128), jnp.int32),
            pltpu.VMEM((SB // 128, 128), jnp.int32),
            pltpu.VMEM((SB // 128, 128), jnp.int32),
            pltpu.VMEM((128, 2 * D), jnp.float32),
            pltpu.VMEM((128, 2 * D), jnp.float32),
            pltpu.VMEM((64, 2 * D), jnp.float32),
        ],
    )


_sc_kernel = _build()


def kernel(mem, idx, val, sample_idx):
    mem2 = mem.reshape(M // 2, 2 * D)
    val2 = val.reshape(B // 2, 2 * D)
    out2 = _sc_kernel(mem2, idx, val2, sample_idx)
    return out2.reshape(B, D)


# speedup vs baseline: 1.0005x; 1.0005x over previous
import jax
import jax.numpy as jnp
from jax import lax
from jax.experimental import pallas as pl
from jax.experimental.pallas import tpu as pltpu
from jax.experimental.pallas import tpu_sc as plsc

M = 1_000_000
D = 64
B = 16384
TBL = 1 << 20
NC, NS = 2, 16
NW = NC * NS
SB = B // NW
IB = B // NS
ROUNDS = 3
FILL = 16384
REG = TBL // NS
CH = 128


def _body(mem2_hbm, idx_hbm, val2_hbm, samp_hbm, out1_hbm,
          tbl_sh, fill_v, idxs_v, jv_v, cur_v, tgt_v,
          samp_v, p_v, pair_v, row_v, col_v, rows_ab, outbuf):
    c = lax.axis_index("c")
    s = lax.axis_index("s")
    wid = s * NC + c
    ii16 = lax.iota(jnp.int32, 16)
    neg1 = jnp.full((16,), -1, jnp.int32)

    def _fill(i, _):
        fill_v[pl.ds(i * 16, 16)] = neg1
        return _
    lax.fori_loop(0, FILL // 16, _fill, 0)
    for b in range(REG // FILL):
        pltpu.sync_copy(fill_v, tbl_sh.at[pl.ds(s * REG + b * FILL, FILL)])

    pltpu.sync_copy(idx_hbm.at[pl.ds(s * IB, IB)], idxs_v)

    def _jv(g, _):
        jv_v[pl.ds(g * 16, 16)] = s * IB + g * 16 + ii16
        return _
    lax.fori_loop(0, IB // 16, _jv, 0)
    plsc.subcore_barrier()

    dump_cell = jnp.full((16,), M, jnp.int32) + wid
    for r in range(ROUNDS):
        if r == 0:
            src = idxs_v
        else:
            pltpu.sync_copy(tbl_sh.at[idxs_v], cur_v)

            def _cmp(g, _):
                cu = cur_v[pl.ds(g * 16, 16)]
                jj = jv_v[pl.ds(g * 16, 16)]
                ix = idxs_v[pl.ds(g * 16, 16)]
                tgt_v[pl.ds(g * 16, 16)] = jnp.where(cu < jj, ix, dump_cell)
                return _
            lax.fori_loop(0, IB // 16, _cmp, 0)
            src = tgt_v
        pltpu.sync_copy(jv_v, tbl_sh.at[src])
        plsc.subcore_barrier()

    pltpu.sync_copy(samp_hbm.at[pl.ds(wid * SB, SB)], samp_v)
    pltpu.sync_copy(tbl_sh.at[samp_v], p_v)
    for q in range(SB // CH):
        def _pidx(g, _):
            sv = samp_v[pl.ds(q * CH + g * 16, 16)]
            pair_v[pl.ds(g * 16, 16)] = lax.shift_right_logical(sv, 1)
            return _
        lax.fori_loop(0, CH // 16, _pidx, 0)
        pltpu.sync_copy(mem2_hbm.at[pair_v], rows_ab.at[pl.ds(0, CH)])

        def _vidx(g, _):
            pv = jnp.maximum(p_v[pl.ds(q * CH + g * 16, 16)], 0)
            pair_v[pl.ds(g * 16, 16)] = lax.shift_right_logical(pv, 1)
            return _
        lax.fori_loop(0, CH // 16, _vidx, 0)
        pltpu.sync_copy(val2_hbm.at[pair_v], rows_ab.at[pl.ds(CH, CH)])

        def _sel(g, _):
            sv = samp_v[pl.ds(q * CH + g * 16, 16)]
            pv = p_v[pl.ds(q * CH + g * 16, 16)]
            hit = pv >= 0
            i16 = g * 16 + ii16
            row_v[pl.ds(g * 16, 16)] = jnp.where(hit, CH + i16, i16)
            col_v[pl.ds(g * 16, 16)] = jnp.where(
                hit, (jnp.maximum(pv, 0) & 1) * 64, (sv & 1) * 64)
            return _
        lax.fori_loop(0, CH // 16, _sel, 0)

        def _asm(i, _):
            i_full = jnp.full((16,), i, jnp.int32)
            r_spl = plsc.load_gather(row_v, [i_full])
            c_spl = plsc.load_gather(col_v, [i_full])
            for t in range(4):
                v = plsc.load_gather(
                    rows_ab, [r_spl, c_spl + t * 16 + ii16])
                outbuf[pl.ds(i * D + t * 16, 16)] = v
            return _
        lax.fori_loop(0, CH, _asm, 0)
        pltpu.sync_copy(outbuf,
                        out1_hbm.at[pl.ds((wid * SB + q * CH) * D, CH * D)])


def _build():
    mesh = plsc.VectorSubcoreMesh(core_axis_name="c", subcore_axis_name="s")
    return pl.kernel(
        _body,
        out_type=jax.ShapeDtypeStruct((B * D,), jnp.float32),
        mesh=mesh,
        compiler_params=pltpu.CompilerParams(needs_layout_passes=False),
        scratch_types=[
            pltpu.VMEM_SHARED((TBL,), jnp.int32),
            pltpu.VMEM((FILL,), jnp.int32),
            pltpu.VMEM((IB,), jnp.int32),
            pltpu.VMEM((IB,), jnp.int32),
            pltpu.VMEM((IB,), jnp.int32),
            pltpu.VMEM((IB,), jnp.int32),
            pltpu.VMEM((SB,), jnp.int32),
            pltpu.VMEM((SB,), jnp.int32),
            pltpu.VMEM((CH,), jnp.int32),
            pltpu.VMEM((CH,), jnp.int32),
            pltpu.VMEM((CH,), jnp.int32),
            pltpu.VMEM((2 * CH, 2 * D), jnp.float32),
            pltpu.VMEM((CH * D,), jnp.float32),
        ],
    )


_sc_kernel = _build()


def kernel(mem, idx, val, sample_idx):
    mem2 = mem.reshape(M // 2, 2 * D)
    val2 = val.reshape(B // 2, 2 * D)
    out1 = _sc_kernel(mem2, idx, val2, sample_idx)
    return out1.reshape(B, D)


# speedup vs baseline: 1.0012x; 1.0007x over previous
import jax
import jax.numpy as jnp
from jax import lax
from jax.experimental import pallas as pl
from jax.experimental.pallas import tpu as pltpu
from jax.experimental.pallas import tpu_sc as plsc

M = 1_000_000
D = 64
B = 16384
TBL = 1 << 20
NC, NS = 2, 16
NW = NC * NS
SB = B // NW
IB = B // NS
ROUNDS = 3
FILL = 16384
REG = TBL // NS
CH = 128


def _body(mem2_hbm, idx_hbm, val2_hbm, samp_hbm, out1_hbm,
          tbl_sh, fill_v, idxs_v, jv_v, cur_v, tgt_v,
          samp_v, p_v, pair_v, row_v, col_v, rows_ab, outbuf):
    c = lax.axis_index("c")
    s = lax.axis_index("s")
    wid = s * NC + c
    ii16 = lax.iota(jnp.int32, 16)
    neg1 = jnp.full((16,), -1, jnp.int32)

    _ns = jax.named_scope
    _s1 = _ns("ph_memset"); _s1.__enter__()
    def _fill(i, _):
        fill_v[pl.ds(i * 16, 16)] = neg1
        return _
    lax.fori_loop(0, FILL // 16, _fill, 0)
    for b in range(REG // FILL):
        pltpu.sync_copy(fill_v, tbl_sh.at[pl.ds(s * REG + b * FILL, FILL)])

    _s1.__exit__(None, None, None)
    _s2 = _ns("ph_stage"); _s2.__enter__()
    pltpu.sync_copy(idx_hbm.at[pl.ds(s * IB, IB)], idxs_v)

    def _jv(g, _):
        jv_v[pl.ds(g * 16, 16)] = s * IB + g * 16 + ii16
        return _
    lax.fori_loop(0, IB // 16, _jv, 0)
    plsc.subcore_barrier()

    _s2.__exit__(None, None, None)
    _s3 = _ns("ph_rounds"); _s3.__enter__()
    dump_cell = jnp.full((16,), M, jnp.int32) + wid
    for r in range(ROUNDS):
        if r == 0:
            src = idxs_v
        else:
            pltpu.sync_copy(tbl_sh.at[idxs_v], cur_v)

            def _cmp(g, _):
                cu = cur_v[pl.ds(g * 16, 16)]
                jj = jv_v[pl.ds(g * 16, 16)]
                ix = idxs_v[pl.ds(g * 16, 16)]
                tgt_v[pl.ds(g * 16, 16)] = jnp.where(cu < jj, ix, dump_cell)
                return _
            lax.fori_loop(0, IB // 16, _cmp, 0)
            src = tgt_v
        pltpu.sync_copy(jv_v, tbl_sh.at[src])
        plsc.subcore_barrier()

    _s3.__exit__(None, None, None)
    _s4 = _ns("ph_b_gather"); _s4.__enter__()
    pltpu.sync_copy(samp_hbm.at[pl.ds(wid * SB, SB)], samp_v)
    pltpu.sync_copy(tbl_sh.at[samp_v], p_v)
    for q in range(SB // CH):
        def _pidx(g, _):
            sv = samp_v[pl.ds(q * CH + g * 16, 16)]
            pair_v[pl.ds(g * 16, 16)] = lax.shift_right_logical(sv, 1)
            return _
        lax.fori_loop(0, CH // 16, _pidx, 0)
        pltpu.sync_copy(mem2_hbm.at[pair_v], rows_ab.at[pl.ds(0, CH)])

        def _vidx(g, _):
            pv = jnp.maximum(p_v[pl.ds(q * CH + g * 16, 16)], 0)
            pair_v[pl.ds(g * 16, 16)] = lax.shift_right_logical(pv, 1)
            return _
        lax.fori_loop(0, CH // 16, _vidx, 0)
        pltpu.sync_copy(val2_hbm.at[pair_v], rows_ab.at[pl.ds(CH, CH)])

        def _sel(g, _):
            sv = samp_v[pl.ds(q * CH + g * 16, 16)]
            pv = p_v[pl.ds(q * CH + g * 16, 16)]
            hit = pv >= 0
            i16 = g * 16 + ii16
            row_v[pl.ds(g * 16, 16)] = jnp.where(hit, CH + i16, i16)
            col_v[pl.ds(g * 16, 16)] = jnp.where(
                hit, (jnp.maximum(pv, 0) & 1) * 64, (sv & 1) * 64)
            return _
        lax.fori_loop(0, CH // 16, _sel, 0)

        _s5 = _ns("ph_asm"); _s5.__enter__()
        def _asm(i, _):
            i_full = jnp.full((16,), i, jnp.int32)
            r_spl = plsc.load_gather(row_v, [i_full])
            c_spl = plsc.load_gather(col_v, [i_full])
            for t in range(4):
                v = plsc.load_gather(
                    rows_ab, [r_spl, c_spl + t * 16 + ii16])
                outbuf[pl.ds(i * D + t * 16, 16)] = v
            return _
        lax.fori_loop(0, CH, _asm, 0)
        _s5.__exit__(None, None, None)
        pltpu.sync_copy(outbuf,
                        out1_hbm.at[pl.ds((wid * SB + q * CH) * D, CH * D)])
    _s4.__exit__(None, None, None)


def _build():
    mesh = plsc.VectorSubcoreMesh(core_axis_name="c", subcore_axis_name="s")
    return pl.kernel(
        _body,
        out_type=jax.ShapeDtypeStruct((B * D,), jnp.float32),
        mesh=mesh,
        compiler_params=pltpu.CompilerParams(needs_layout_passes=False),
        scratch_types=[
            pltpu.VMEM_SHARED((TBL,), jnp.int32),
            pltpu.VMEM((FILL,), jnp.int32),
            pltpu.VMEM((IB,), jnp.int32),
            pltpu.VMEM((IB,), jnp.int32),
            pltpu.VMEM((IB,), jnp.int32),
            pltpu.VMEM((IB,), jnp.int32),
            pltpu.VMEM((SB,), jnp.int32),
            pltpu.VMEM((SB,), jnp.int32),
            pltpu.VMEM((CH,), jnp.int32),
            pltpu.VMEM((CH,), jnp.int32),
            pltpu.VMEM((CH,), jnp.int32),
            pltpu.VMEM((2 * CH, 2 * D), jnp.float32),
            pltpu.VMEM((CH * D,), jnp.float32),
        ],
    )


_sc_kernel = _build()


def kernel(mem, idx, val, sample_idx):
    mem2 = mem.reshape(M // 2, 2 * D)
    val2 = val.reshape(B // 2, 2 * D)
    out1 = _sc_kernel(mem2, idx, val2, sample_idx)
    return out1.reshape(B, D)


# speedup vs baseline: 3.3148x; 3.3107x over previous
import jax
import jax.numpy as jnp
from jax import lax
from jax.experimental import pallas as pl
from jax.experimental.pallas import tpu as pltpu
from jax.experimental.pallas import tpu_sc as plsc

M = 1_000_000
D = 64
B = 16384
TBL = 1 << 20
NC, NS = 2, 16
NW = NC * NS
SB = B // NW
IB = B // NS
ROUNDS = 3
FILL = 8192
REG = TBL // NS
NSEM = 8
HB = 256
RPS = HB // NSEM


def _body(mem_hbm, idx_hbm, val_hbm, samp_hbm, out_hbm,
          tbl_sh, fill_v, idxs_v, jv_v, cur_v, tgt_v,
          samp_v, p_v, rows_v, *sems):
    c = lax.axis_index("c")
    s = lax.axis_index("s")
    wid = s * NC + c
    ii16 = lax.iota(jnp.int32, 16)
    neg1 = jnp.full((16,), -1, jnp.int32)

    with jax.named_scope("ph_memset"):
        def _fill(i, _):
            fill_v[pl.ds(i * 16, 16)] = neg1
            return _
        lax.fori_loop(0, FILL // 16, _fill, 0)
        for b in range(REG // FILL):
            pltpu.sync_copy(fill_v,
                            tbl_sh.at[pl.ds(s * REG + b * FILL, FILL)])
        pltpu.sync_copy(idx_hbm.at[pl.ds(s * IB, IB)], idxs_v)

        def _jv(g, _):
            jv_v[pl.ds(g * 16, 16)] = s * IB + g * 16 + ii16
            return _
        lax.fori_loop(0, IB // 16, _jv, 0)
        plsc.subcore_barrier()

    with jax.named_scope("ph_rounds"):
        dump_cell = jnp.full((16,), M, jnp.int32) + wid
        for r in range(ROUNDS):
            if r == 0:
                src = idxs_v
            else:
                pltpu.sync_copy(tbl_sh.at[idxs_v], cur_v)

                def _cmp(g, _):
                    cu = cur_v[pl.ds(g * 16, 16)]
                    jj = jv_v[pl.ds(g * 16, 16)]
                    ix = idxs_v[pl.ds(g * 16, 16)]
                    tgt_v[pl.ds(g * 16, 16)] = jnp.where(
                        cu < jj, ix, dump_cell)
                    return _
                lax.fori_loop(0, IB // 16, _cmp, 0)
                src = tgt_v
            pltpu.sync_copy(jv_v, tbl_sh.at[src])
            plsc.subcore_barrier()

    with jax.named_scope("ph_b"):
        pltpu.sync_copy(samp_hbm.at[pl.ds(wid * SB, SB)], samp_v)
        pltpu.sync_copy(tbl_sh.at[samp_v], p_v)
        for h in range(SB // HB):
            for b in range(NSEM):
                sem = sems[b]

                def _grp(g, _):
                    base = h * HB + b * RPS + g * 16
                    sv16 = samp_v[pl.ds(base, 16)]
                    pv16 = p_v[pl.ds(base, 16)]
                    for l in range(16):
                        sv = sv16[l]
                        pv = pv16[l]
                        o = b * RPS + g * 16 + l

                        @pl.when(pv >= 0)
                        def _hit():
                            pltpu.async_copy(val_hbm.at[pv], rows_v.at[o],
                                             sem)

                        @pl.when(pv < 0)
                        def _miss():
                            pltpu.async_copy(mem_hbm.at[sv], rows_v.at[o],
                                             sem)

                    return _
                lax.fori_loop(0, RPS // 16, _grp, 0)
            for b in range(NSEM):
                pltpu.make_async_copy(
                    mem_hbm.at[pl.ds(0, RPS)],
                    rows_v.at[pl.ds(b * RPS, RPS)], sems[b]).wait()
            pltpu.sync_copy(rows_v,
                            out_hbm.at[pl.ds(wid * SB + h * HB, HB)])


def _build():
    mesh = plsc.VectorSubcoreMesh(core_axis_name="c", subcore_axis_name="s")
    return pl.kernel(
        _body,
        out_type=jax.ShapeDtypeStruct((B, D), jnp.float32),
        mesh=mesh,
        compiler_params=pltpu.CompilerParams(needs_layout_passes=False),
        scratch_types=[
            pltpu.VMEM_SHARED((TBL,), jnp.int32),
            pltpu.VMEM((FILL,), jnp.int32),
            pltpu.VMEM((IB,), jnp.int32),
            pltpu.VMEM((IB,), jnp.int32),
            pltpu.VMEM((IB,), jnp.int32),
            pltpu.VMEM((IB,), jnp.int32),
            pltpu.VMEM((SB,), jnp.int32),
            pltpu.VMEM((SB,), jnp.int32),
            pltpu.VMEM((HB, D), jnp.float32),
        ] + [pltpu.SemaphoreType.DMA] * NSEM,
    )


_sc_kernel = _build()


def kernel(mem, idx, val, sample_idx):
    return _sc_kernel(mem, idx, val, sample_idx)
